# baseline (device time: 36331 ns/iter reference)
import jax
import jax.numpy as jnp
from jax import lax
from jax.experimental import pallas as pl
from jax.experimental.pallas import tpu as pltpu

N_DEV = 16
B, SQ, SKV, HL, DH = 2, 256, 256, 4, 64
DM = 512
DP = HL * DH
HIDDEN = N_DEV * DP
WIN = 128
NCHUNK = N_DEV
CROWS = (B * SQ) // NCHUNK
CPB = SQ // CROWS


def kernel(x, Wq, K_ext, V_ext, Wo):
    def body(x_ref, wq_ref, k_ref, v_ref, wo_ref, out_ref,
             wq_s, wo_s, ctxb, comm, dma_sems, send_sems, rs_sems, ag_sems):
        p = lax.axis_index("i")

        barrier = pltpu.get_barrier_semaphore()
        for d in range(1, N_DEV):
            nbr = lax.rem(p + d, N_DEV)
            pl.semaphore_signal(
                barrier, inc=1,
                device_id=(nbr,), device_id_type=pl.DeviceIdType.MESH,
            )

        cq = pltpu.make_async_copy(
            wq_ref.at[:, pl.ds(p * DP, DP)], wq_s, dma_sems.at[0])
        co = pltpu.make_async_copy(wo_ref, wo_s, dma_sems.at[1])
        cq.start()
        co.start()
        cq.wait()

        qi = lax.broadcasted_iota(jnp.int32, (SQ, SKV), 0)
        ki = lax.broadcasted_iota(jnp.int32, (SQ, SKV), 1)
        mask = jnp.abs(qi - ki) <= WIN

        def attend(b):
            q_b = jnp.dot(x_ref[b], wq_s[...],
                          preferred_element_type=jnp.float32)
            for h in range(HL):
                q_h = q_b[:, h * DH:(h + 1) * DH]
                k_h = k_ref[b, :, h, :]
                v_h = v_ref[b, :, h, :]
                s = lax.dot_general(
                    q_h, k_h, (((1,), (1,)), ((), ())),
                    preferred_element_type=jnp.float32) * 0.125
                s = jnp.where(mask, s, -1e9)
                m = jnp.max(s, axis=1, keepdims=True)
                w = jnp.exp(s - m)
                w = w / jnp.sum(w, axis=1, keepdims=True)
                ctx_h = jnp.dot(w, v_h, preferred_element_type=jnp.float32)
                ctxb[pl.ds(b * SQ, SQ), h * DH:(h + 1) * DH] = ctx_h

        def rs_send(c):
            send = pltpu.make_async_remote_copy(
                src_ref=ctxb.at[pl.ds(c * CROWS, CROWS), :],
                dst_ref=comm.at[p],
                send_sem=send_sems.at[c],
                recv_sem=rs_sems.at[p],
                device_id=(c,), device_id_type=pl.DeviceIdType.MESH,
            )
            send.start()

        attend(0)
        pl.semaphore_wait(barrier, N_DEV - 1)
        for d in range(1, N_DEV):
            c = lax.rem(p + d, N_DEV)

            @pl.when(c < CPB)
            def _():
                rs_send(c)

        attend(1)
        for d in range(1, N_DEV):
            c = lax.rem(p + d, N_DEV)

            @pl.when(c >= CPB)
            def _():
                rs_send(c)

        comm[p, :, :] = ctxb[pl.ds(p * CROWS, CROWS), :]
        co.wait()

        red = jnp.dot(comm[p], wo_s[pl.ds(p * DP, DP), :],
                      preferred_element_type=jnp.float32)
        for d in range(1, N_DEV):
            q = lax.rem(p + d, N_DEV)
            recv = pltpu.make_async_remote_copy(
                src_ref=comm.at[q],
                dst_ref=comm.at[q],
                send_sem=dma_sems.at[0],
                recv_sem=rs_sems.at[q],
                device_id=(q,), device_id_type=pl.DeviceIdType.MESH,
            )
            recv.wait_recv()
            red = red + jnp.dot(comm[q], wo_s[pl.ds(q * DP, DP), :],
                                preferred_element_type=jnp.float32)
        out_ref[pl.ds(p * CROWS, CROWS), :] = red

        for d in range(1, N_DEV):
            c = lax.rem(p + d, N_DEV)
            pltpu.make_async_remote_copy(
                src_ref=ctxb.at[pl.ds(c * CROWS, CROWS), :],
                dst_ref=comm.at[p],
                send_sem=send_sems.at[c],
                recv_sem=rs_sems.at[p],
                device_id=(c,), device_id_type=pl.DeviceIdType.MESH,
            ).wait_send()

        for d in range(1, N_DEV):
            tgt = lax.rem(p + d, N_DEV)
            send = pltpu.make_async_remote_copy(
                src_ref=out_ref.at[pl.ds(p * CROWS, CROWS), :],
                dst_ref=out_ref.at[pl.ds(p * CROWS, CROWS), :],
                send_sem=send_sems.at[tgt],
                recv_sem=ag_sems.at[p],
                device_id=(tgt,), device_id_type=pl.DeviceIdType.MESH,
            )
            send.start()

        for d in range(1, N_DEV):
            c = lax.rem(p + d, N_DEV)
            recv = pltpu.make_async_remote_copy(
                src_ref=out_ref.at[pl.ds(c * CROWS, CROWS), :],
                dst_ref=out_ref.at[pl.ds(c * CROWS, CROWS), :],
                send_sem=dma_sems.at[0],
                recv_sem=ag_sems.at[c],
                device_id=(c,), device_id_type=pl.DeviceIdType.MESH,
            )
            recv.wait_recv()

        for d in range(1, N_DEV):
            tgt = lax.rem(p + d, N_DEV)
            pltpu.make_async_remote_copy(
                src_ref=out_ref.at[pl.ds(p * CROWS, CROWS), :],
                dst_ref=out_ref.at[pl.ds(p * CROWS, CROWS), :],
                send_sem=send_sems.at[tgt],
                recv_sem=ag_sems.at[p],
                device_id=(tgt,), device_id_type=pl.DeviceIdType.MESH,
            ).wait_send()

    out_flat = pl.pallas_call(
        body,
        out_shape=jax.ShapeDtypeStruct((B * SQ, DM), jnp.float32),
        in_specs=[
            pl.BlockSpec(memory_space=pltpu.MemorySpace.VMEM),
            pl.BlockSpec(memory_space=pl.ANY),
            pl.BlockSpec(memory_space=pltpu.MemorySpace.VMEM),
            pl.BlockSpec(memory_space=pltpu.MemorySpace.VMEM),
            pl.BlockSpec(memory_space=pl.ANY),
        ],
        out_specs=pl.BlockSpec(memory_space=pltpu.MemorySpace.VMEM),
        scratch_shapes=[
            pltpu.VMEM((DM, DP), jnp.float32),
            pltpu.VMEM((HIDDEN, DM), jnp.float32),
            pltpu.VMEM((B * SQ, DP), jnp.float32),
            pltpu.VMEM((NCHUNK, CROWS, DP), jnp.float32),
            pltpu.SemaphoreType.DMA((2,)),
            pltpu.SemaphoreType.DMA((NCHUNK,)),
            pltpu.SemaphoreType.DMA((NCHUNK,)),
            pltpu.SemaphoreType.DMA((NCHUNK,)),
        ],
        compiler_params=pltpu.CompilerParams(collective_id=0),
    )(x, Wq, K_ext, V_ext, Wo)
    return out_flat.reshape(B, SQ, DM)


# device time: 33884 ns/iter; 1.0722x vs baseline; 1.0722x over previous
import jax
import jax.numpy as jnp
from jax import lax
from jax.experimental import pallas as pl
from jax.experimental.pallas import tpu as pltpu

N_DEV = 16
B, SQ, SKV, HL, DH = 2, 256, 256, 4, 64
DM = 512
DP = HL * DH
WIN = 128
NCHUNK = N_DEV
CROWS = (B * SQ) // NCHUNK
CPB = SQ // CROWS
BF = jnp.bfloat16
F32 = jnp.float32


def kernel(x, Wq, K_ext, V_ext, Wo):
    def body(x_ref, wq_ref, k_ref, v_ref, wo_ref, out_ref,
             wq_s, wo_s, acc, comm, outb,
             dma_sems, send_sems, rs_sems, ag_sems):
        p = lax.axis_index("i")

        barrier = pltpu.get_barrier_semaphore()
        for d in range(1, N_DEV):
            nbr = lax.rem(p + d, N_DEV)
            pl.semaphore_signal(
                barrier, inc=1,
                device_id=(nbr,), device_id_type=pl.DeviceIdType.MESH,
            )

        cq = pltpu.make_async_copy(
            wq_ref.at[:, pl.ds(p * DP, DP)], wq_s, dma_sems.at[0])
        co = pltpu.make_async_copy(
            wo_ref.at[pl.ds(p * DP, DP), :], wo_s, dma_sems.at[1])
        cq.start()
        co.start()
        cq.wait()
        co.wait()

        wq_bf = wq_s[...].astype(BF)
        wo_bf = wo_s[...].astype(BF)
        qi = lax.broadcasted_iota(jnp.int32, (SQ, SKV), 0)
        ki = lax.broadcasted_iota(jnp.int32, (SQ, SKV), 1)
        mask = jnp.abs(qi - ki) <= WIN

        def attend(b):
            q_b = jnp.dot(x_ref[b].astype(BF), wq_bf,
                          preferred_element_type=F32)
            part = jnp.zeros((SQ, DM), F32)
            for h in range(HL):
                q_h = q_b[:, h * DH:(h + 1) * DH].astype(BF)
                k_h = k_ref[b, :, h, :].astype(BF)
                v_h = v_ref[b, :, h, :].astype(BF)
                s = lax.dot_general(
                    q_h, k_h, (((1,), (1,)), ((), ())),
                    preferred_element_type=F32) * 0.125
                s = jnp.where(mask, s, -1e9)
                m = jnp.max(s, axis=1, keepdims=True)
                w = jnp.exp(s - m)
                w = (w / jnp.sum(w, axis=1, keepdims=True)).astype(BF)
                ctx_h = jnp.dot(w, v_h, preferred_element_type=F32)
                part = part + jnp.dot(
                    ctx_h.astype(BF), wo_bf[h * DH:(h + 1) * DH, :],
                    preferred_element_type=F32)
            acc[pl.ds(b * SQ, SQ), :] = part.astype(BF)

        def rs_send(c):
            pltpu.make_async_remote_copy(
                src_ref=acc.at[pl.ds(c * CROWS, CROWS), :],
                dst_ref=comm.at[p],
                send_sem=send_sems.at[c],
                recv_sem=rs_sems.at[p],
                device_id=(c,), device_id_type=pl.DeviceIdType.MESH,
            ).start()

        attend(0)
        pl.semaphore_wait(barrier, N_DEV - 1)
        for d in range(1, N_DEV):
            c = lax.rem(p + d, N_DEV)

            @pl.when(c < CPB)
            def _():
                rs_send(c)

        attend(1)
        for d in range(1, N_DEV):
            c = lax.rem(p + d, N_DEV)

            @pl.when(c >= CPB)
            def _():
                rs_send(c)

        red = acc[pl.ds(p * CROWS, CROWS), :].astype(F32)
        for d in range(1, N_DEV):
            q = lax.rem(p + d, N_DEV)
            recv = pltpu.make_async_remote_copy(
                src_ref=comm.at[q],
                dst_ref=comm.at[q],
                send_sem=dma_sems.at[0],
                recv_sem=rs_sems.at[q],
                device_id=(q,), device_id_type=pl.DeviceIdType.MESH,
            )
            recv.wait_recv()
            red = red + comm[q].astype(F32)
        out_ref[pl.ds(p * CROWS, CROWS), :] = red
        outb[pl.ds(p * CROWS, CROWS), :] = red.astype(BF)

        for d in range(1, N_DEV):
            c = lax.rem(p + d, N_DEV)
            pltpu.make_async_remote_copy(
                src_ref=acc.at[pl.ds(c * CROWS, CROWS), :],
                dst_ref=comm.at[p],
                send_sem=send_sems.at[c],
                recv_sem=rs_sems.at[p],
                device_id=(c,), device_id_type=pl.DeviceIdType.MESH,
            ).wait_send()

        for d in range(1, N_DEV):
            tgt = lax.rem(p + d, N_DEV)
            pltpu.make_async_remote_copy(
                src_ref=outb.at[pl.ds(p * CROWS, CROWS), :],
                dst_ref=outb.at[pl.ds(p * CROWS, CROWS), :],
                send_sem=send_sems.at[tgt],
                recv_sem=ag_sems.at[p],
                device_id=(tgt,), device_id_type=pl.DeviceIdType.MESH,
            ).start()

        for d in range(1, N_DEV):
            c = lax.rem(p + d, N_DEV)
            recv = pltpu.make_async_remote_copy(
                src_ref=outb.at[pl.ds(c * CROWS, CROWS), :],
                dst_ref=outb.at[pl.ds(c * CROWS, CROWS), :],
                send_sem=dma_sems.at[0],
                recv_sem=ag_sems.at[c],
                device_id=(c,), device_id_type=pl.DeviceIdType.MESH,
            )
            recv.wait_recv()
            out_ref[pl.ds(c * CROWS, CROWS), :] = (
                outb[pl.ds(c * CROWS, CROWS), :].astype(F32))

        for d in range(1, N_DEV):
            tgt = lax.rem(p + d, N_DEV)
            pltpu.make_async_remote_copy(
                src_ref=outb.at[pl.ds(p * CROWS, CROWS), :],
                dst_ref=outb.at[pl.ds(p * CROWS, CROWS), :],
                send_sem=send_sems.at[tgt],
                recv_sem=ag_sems.at[p],
                device_id=(tgt,), device_id_type=pl.DeviceIdType.MESH,
            ).wait_send()

    out_flat = pl.pallas_call(
        body,
        out_shape=jax.ShapeDtypeStruct((B * SQ, DM), F32),
        in_specs=[
            pl.BlockSpec(memory_space=pltpu.MemorySpace.VMEM),
            pl.BlockSpec(memory_space=pl.ANY),
            pl.BlockSpec(memory_space=pltpu.MemorySpace.VMEM),
            pl.BlockSpec(memory_space=pltpu.MemorySpace.VMEM),
            pl.BlockSpec(memory_space=pl.ANY),
        ],
        out_specs=pl.BlockSpec(memory_space=pltpu.MemorySpace.VMEM),
        scratch_shapes=[
            pltpu.VMEM((DM, DP), F32),
            pltpu.VMEM((DP, DM), F32),
            pltpu.VMEM((B * SQ, DM), BF),
            pltpu.VMEM((NCHUNK, CROWS, DM), BF),
            pltpu.VMEM((B * SQ, DM), BF),
            pltpu.SemaphoreType.DMA((2,)),
            pltpu.SemaphoreType.DMA((NCHUNK,)),
            pltpu.SemaphoreType.DMA((NCHUNK,)),
            pltpu.SemaphoreType.DMA((NCHUNK,)),
        ],
        compiler_params=pltpu.CompilerParams(collective_id=0),
    )(x, Wq, K_ext, V_ext, Wo)
    return out_flat.reshape(B, SQ, DM)


# device time: 31167 ns/iter; 1.1657x vs baseline; 1.0872x over previous
import jax
import jax.numpy as jnp
from jax import lax
from jax.experimental import pallas as pl
from jax.experimental.pallas import tpu as pltpu

N_DEV = 16
B, SQ, SKV, HL, DH = 2, 256, 256, 4, 64
DM = 512
DP = HL * DH
WIN = 128
NCHUNK = N_DEV
CROWS = (B * SQ) // NCHUNK
CPB = SQ // CROWS
BF = jnp.bfloat16
F32 = jnp.float32


def kernel(x, Wq, K_ext, V_ext, Wo):
    def body(x_ref, wq_ref, k_ref, v_ref, wo_ref, out_ref,
             wq_s, wo_s, acc, comm, outb,
             dma_sems, send_sems, rs_sems, ag_sems):
        p = lax.axis_index("i")

        barrier = pltpu.get_barrier_semaphore()
        for d in range(1, N_DEV):
            nbr = lax.rem(p + d, N_DEV)
            pl.semaphore_signal(
                barrier, inc=1,
                device_id=(nbr,), device_id_type=pl.DeviceIdType.MESH,
            )

        cq = pltpu.make_async_copy(
            wq_ref.at[:, pl.ds(p * DP, DP)], wq_s, dma_sems.at[0])
        co = pltpu.make_async_copy(
            wo_ref.at[pl.ds(p * DP, DP), :], wo_s, dma_sems.at[1])
        cq.start()
        co.start()
        cq.wait()
        co.wait()

        wq_bf = (wq_s[...] * 0.125).astype(BF)
        wo_bf = wo_s[...].astype(BF)
        qi = lax.broadcasted_iota(jnp.int32, (SQ, SKV), 0)
        ki = lax.broadcasted_iota(jnp.int32, (SQ, SKV), 1)
        mask = jnp.abs(qi - ki) <= WIN

        def attend(b):
            q_b = jnp.dot(x_ref[b].astype(BF), wq_bf,
                          preferred_element_type=F32)
            q_hsd = q_b.reshape(SQ, HL, DH).swapaxes(0, 1).astype(BF)
            k_hsd = k_ref[b].astype(BF).swapaxes(0, 1)
            v_hsd = v_ref[b].astype(BF).swapaxes(0, 1)
            s = lax.dot_general(
                q_hsd, k_hsd, (((2,), (2,)), ((0,), (0,))),
                preferred_element_type=F32)
            w = jnp.exp(jnp.where(mask[None], s, -1e9))
            w = (w * (1.0 / jnp.sum(w, axis=2, keepdims=True))).astype(BF)
            ctx = lax.dot_general(
                w, v_hsd, (((2,), (1,)), ((0,), (0,))),
                preferred_element_type=F32)
            part = jnp.zeros((SQ, DM), F32)
            for h in range(HL):
                part = part + jnp.dot(
                    ctx[h].astype(BF), wo_bf[h * DH:(h + 1) * DH, :],
                    preferred_element_type=F32)
            acc[pl.ds(b * SQ, SQ), :] = part.astype(BF)

        def rs_send(c):
            pltpu.make_async_remote_copy(
                src_ref=acc.at[pl.ds(c * CROWS, CROWS), :],
                dst_ref=comm.at[p],
                send_sem=send_sems.at[c],
                recv_sem=rs_sems.at[p],
                device_id=(c,), device_id_type=pl.DeviceIdType.MESH,
            ).start()

        def rs_send_batch(bc):
            for d in range(1, N_DEV):
                c = lax.rem(p + d, N_DEV)

                @pl.when((c >= bc * CPB) & (c < (bc + 1) * CPB))
                def _():
                    rs_send(c)

        @pl.when(p < CPB)
        def _():
            attend(1)
            pl.semaphore_wait(barrier, N_DEV - 1)
            rs_send_batch(1)
            attend(0)
            rs_send_batch(0)

        @pl.when(p >= CPB)
        def _():
            attend(0)
            pl.semaphore_wait(barrier, N_DEV - 1)
            rs_send_batch(0)
            attend(1)
            rs_send_batch(1)

        red = acc[pl.ds(p * CROWS, CROWS), :].astype(F32)
        for d in range(1, N_DEV):
            q = lax.rem(p + d, N_DEV)
            recv = pltpu.make_async_remote_copy(
                src_ref=comm.at[q],
                dst_ref=comm.at[q],
                send_sem=dma_sems.at[0],
                recv_sem=rs_sems.at[q],
                device_id=(q,), device_id_type=pl.DeviceIdType.MESH,
            )
            recv.wait_recv()
            red = red + comm[q].astype(F32)
        out_ref[pl.ds(p * CROWS, CROWS), :] = red
        outb[pl.ds(p * CROWS, CROWS), :] = red.astype(BF)

        for d in range(1, N_DEV):
            c = lax.rem(p + d, N_DEV)
            pltpu.make_async_remote_copy(
                src_ref=acc.at[pl.ds(c * CROWS, CROWS), :],
                dst_ref=comm.at[p],
                send_sem=send_sems.at[c],
                recv_sem=rs_sems.at[p],
                device_id=(c,), device_id_type=pl.DeviceIdType.MESH,
            ).wait_send()

        for d in range(1, N_DEV):
            tgt = lax.rem(p + d, N_DEV)
            pltpu.make_async_remote_copy(
                src_ref=outb.at[pl.ds(p * CROWS, CROWS), :],
                dst_ref=outb.at[pl.ds(p * CROWS, CROWS), :],
                send_sem=send_sems.at[tgt],
                recv_sem=ag_sems.at[p],
                device_id=(tgt,), device_id_type=pl.DeviceIdType.MESH,
            ).start()

        for d in range(1, N_DEV):
            c = lax.rem(p + d, N_DEV)
            recv = pltpu.make_async_remote_copy(
                src_ref=outb.at[pl.ds(c * CROWS, CROWS), :],
                dst_ref=outb.at[pl.ds(c * CROWS, CROWS), :],
                send_sem=dma_sems.at[0],
                recv_sem=ag_sems.at[c],
                device_id=(c,), device_id_type=pl.DeviceIdType.MESH,
            )
            recv.wait_recv()
            out_ref[pl.ds(c * CROWS, CROWS), :] = (
                outb[pl.ds(c * CROWS, CROWS), :].astype(F32))

        for d in range(1, N_DEV):
            tgt = lax.rem(p + d, N_DEV)
            pltpu.make_async_remote_copy(
                src_ref=outb.at[pl.ds(p * CROWS, CROWS), :],
                dst_ref=outb.at[pl.ds(p * CROWS, CROWS), :],
                send_sem=send_sems.at[tgt],
                recv_sem=ag_sems.at[p],
                device_id=(tgt,), device_id_type=pl.DeviceIdType.MESH,
            ).wait_send()

    out_flat = pl.pallas_call(
        body,
        out_shape=jax.ShapeDtypeStruct((B * SQ, DM), F32),
        in_specs=[
            pl.BlockSpec(memory_space=pltpu.MemorySpace.VMEM),
            pl.BlockSpec(memory_space=pl.ANY),
            pl.BlockSpec(memory_space=pltpu.MemorySpace.VMEM),
            pl.BlockSpec(memory_space=pltpu.MemorySpace.VMEM),
            pl.BlockSpec(memory_space=pl.ANY),
        ],
        out_specs=pl.BlockSpec(memory_space=pltpu.MemorySpace.VMEM),
        scratch_shapes=[
            pltpu.VMEM((DM, DP), F32),
            pltpu.VMEM((DP, DM), F32),
            pltpu.VMEM((B * SQ, DM), BF),
            pltpu.VMEM((NCHUNK, CROWS, DM), BF),
            pltpu.VMEM((B * SQ, DM), BF),
            pltpu.SemaphoreType.DMA((2,)),
            pltpu.SemaphoreType.DMA((NCHUNK,)),
            pltpu.SemaphoreType.DMA((NCHUNK,)),
            pltpu.SemaphoreType.DMA((NCHUNK,)),
        ],
        compiler_params=pltpu.CompilerParams(collective_id=0),
    )(x, Wq, K_ext, V_ext, Wo)
    return out_flat.reshape(B, SQ, DM)
